# baseline (speedup 1.0000x reference)
import functools

import jax
import jax.numpy as jnp
from jax import lax
from jax.experimental import pallas as pl
from jax.experimental.pallas import tpu as pltpu
from jax.experimental.pallas import tpu_sc as plsc

E = 160000
N_NODES = 10000
H = 128
H2 = 2 * H
NB_MESS = 6
NB_NODE = 16

NC = 2
NS = 16
NW = NC * NS

BE = 2000
BN = 1000
CE = 128
CN = 80
TE = E // CE
TN = N_NODES // CN

_f32 = jnp.float32


def _row_mask(block_rows, prog_id):
    row = lax.broadcasted_iota(jnp.int32, (block_rows, 1), 0) + prog_id * block_rows
    return (row != 0).astype(_f32)


def _pre_body(fm, wz1, bz, wh1, bh, wr, br, ur, xz_o, xh_o, r1_o, hhu_o):
    fm_ = fm[...]
    xz = jnp.dot(fm_, wz1[...], preferred_element_type=_f32) + bz[...]
    xh = jnp.dot(fm_, wh1[...], preferred_element_type=_f32) + bh[...]
    r1 = jnp.dot(fm_, wr[...], preferred_element_type=_f32) + br[...]
    h1 = jax.nn.sigmoid(xz) * jnp.tanh(xh) * _row_mask(BE, pl.program_id(0))
    xz_o[...] = xz
    xh_o[...] = xh
    r1_o[...] = r1
    hu1 = jnp.dot(h1, ur[...], preferred_element_type=_f32)
    hhu_o[...] = jnp.concatenate([h1, hu1], axis=1)


_rows = pl.BlockSpec((BE, H), lambda i: (i, 0))
_rows2 = pl.BlockSpec((BE, H2), lambda i: (i, 0))
_full = pl.BlockSpec((H, H), lambda i: (0, 0))
_bias = pl.BlockSpec((1, H), lambda i: (0, 0))
_eSDS = jax.ShapeDtypeStruct((E, H), _f32)
_eSDS2 = jax.ShapeDtypeStruct((E, H2), _f32)

_pre_call = pl.pallas_call(
    _pre_body,
    grid=(E // BE,),
    in_specs=[_rows, _full, _bias, _full, _bias, _full, _bias, _full],
    out_specs=[_rows, _rows, _rows, _rows2],
    out_shape=[_eSDS, _eSDS, _eSDS, _eSDS2],
)


def _upd_body(want_hu, sh, sg, xz, xh, wz2, wh2, ur, out):
    sh_ = sh[...]
    z = jax.nn.sigmoid(xz[...] + jnp.dot(sh_, wz2[...], preferred_element_type=_f32))
    p = jnp.tanh(xh[...] + jnp.dot(sg[...], wh2[...], preferred_element_type=_f32))
    hnew = ((1.0 - z) * sh_ + z * p) * _row_mask(BE, pl.program_id(0))
    if want_hu:
        hu = jnp.dot(hnew, ur[...], preferred_element_type=_f32)
        out[...] = jnp.concatenate([hnew, hu], axis=1)
    else:
        out[...] = hnew


_upd_hu = pl.pallas_call(
    functools.partial(_upd_body, True),
    grid=(E // BE,),
    in_specs=[_rows] * 4 + [_full] * 3,
    out_specs=_rows2,
    out_shape=_eSDS2,
)

_upd_last = pl.pallas_call(
    functools.partial(_upd_body, False),
    grid=(E // BE,),
    in_specs=[_rows] * 4 + [_full] * 3,
    out_specs=_rows,
    out_shape=_eSDS,
)


def _out_body(fn, nei, wo1, wo2, bo, o):
    a = (jnp.dot(fn[...], wo1[...], preferred_element_type=_f32)
         + jnp.dot(nei[...], wo2[...], preferred_element_type=_f32) + bo[...])
    o[...] = jnp.maximum(a, 0.0) * _row_mask(BN, pl.program_id(0))


_nrows = pl.BlockSpec((BN, H), lambda i: (i, 0))
_out_call = pl.pallas_call(
    _out_body,
    grid=(N_NODES // BN,),
    in_specs=[_nrows, _nrows, _full, _full, _bias],
    out_specs=_nrows,
    out_shape=jax.ShapeDtypeStruct((N_NODES, H), _f32),
)


@functools.cache
def _make_sc_kernels():
    mesh = plsc.VectorSubcoreMesh(
        core_axis_name="c", subcore_axis_name="s", num_cores=NC, num_subcores=NS)

    @functools.partial(
        pl.kernel,
        out_type=[_eSDS, _eSDS],
        mesh=mesh,
        scratch_types=[
            pltpu.VMEM((NB_MESS * CE,), jnp.int32),
            pltpu.VMEM((CE, H2), _f32),
            pltpu.VMEM((CE, H2), _f32),
            pltpu.VMEM((CE, H), _f32),
            pltpu.VMEM((CE, H), _f32),
            pltpu.VMEM((CE, H), _f32),
            pltpu.SemaphoreType.DMA,
            pltpu.SemaphoreType.DMA,
            pltpu.SemaphoreType.DMA,
        ],
    )
    def sc_edge(hhu_hbm, r1_hbm, bg_hbm, sh_hbm, sg_hbm,
                idx_v, hh0, hh1, r1v, accs, accg, sem0, sem1, sem_r):
        wid = lax.axis_index("s") * NC + lax.axis_index("c")
        bufs = (hh0, hh1)
        sems = (sem0, sem1)

        def chunk(i, carry):
            t = i * NW + wid

            @pl.when(t < TE)
            def _():
                base = t * CE
                pltpu.sync_copy(bg_hbm.at[pl.ds(t * (NB_MESS * CE), NB_MESS * CE)], idx_v)
                dr = pltpu.async_copy(r1_hbm.at[pl.ds(base, CE)], r1v, sem_r)
                pend = pltpu.async_copy(hhu_hbm.at[idx_v.at[pl.ds(0, CE)]], bufs[0], sems[0])
                dr.wait()
                for k in range(NB_MESS):
                    cur = k % 2
                    if k + 1 < NB_MESS:
                        nxt = pltpu.async_copy(
                            hhu_hbm.at[idx_v.at[pl.ds((k + 1) * CE, CE)]], bufs[1 - cur], sems[1 - cur])
                    pend.wait()
                    hh = bufs[cur]

                    def row_body(r, rc):
                        for j in range(H // 16):
                            sl = pl.ds(j * 16, 16)
                            hv = hh[r, sl]
                            x = r1v[r, sl] + hh[r, pl.ds(H + j * 16, 16)]
                            s = 1.0 / (1.0 + jnp.exp(-x))
                            if k == 0:
                                accs[r, sl] = hv
                                accg[r, sl] = s * hv
                            else:
                                accs[r, sl] = accs[r, sl] + hv
                                accg[r, sl] = accg[r, sl] + s * hv
                        return rc

                    lax.fori_loop(0, CE, row_body, 0)
                    if k + 1 < NB_MESS:
                        pend = nxt
                pltpu.sync_copy(accs, sh_hbm.at[pl.ds(base, CE)])
                pltpu.sync_copy(accg, sg_hbm.at[pl.ds(base, CE)])

            return carry

        lax.fori_loop(0, (TE + NW - 1) // NW, chunk, 0)

    @functools.partial(
        pl.kernel,
        out_type=jax.ShapeDtypeStruct((N_NODES, H), _f32),
        mesh=mesh,
        scratch_types=[
            pltpu.VMEM((NB_NODE * CN,), jnp.int32),
            pltpu.VMEM((CN, H), _f32),
            pltpu.VMEM((CN, H), _f32),
            pltpu.VMEM((CN, H), _f32),
            pltpu.SemaphoreType.DMA,
            pltpu.SemaphoreType.DMA,
        ],
    )
    def sc_node(h_hbm, ag_hbm, nei_hbm, idx_v, rw0, rw1, acc, sem0, sem1):
        wid = lax.axis_index("s") * NC + lax.axis_index("c")
        bufs = (rw0, rw1)
        sems = (sem0, sem1)

        def chunk(i, carry):
            t = i * NW + wid

            @pl.when(t < TN)
            def _():
                base = t * CN
                pltpu.sync_copy(ag_hbm.at[pl.ds(t * (NB_NODE * CN), NB_NODE * CN)], idx_v)
                pend = pltpu.async_copy(h_hbm.at[idx_v.at[pl.ds(0, CN)]], bufs[0], sems[0])
                for k in range(NB_NODE):
                    cur = k % 2
                    if k + 1 < NB_NODE:
                        nxt = pltpu.async_copy(
                            h_hbm.at[idx_v.at[pl.ds((k + 1) * CN, CN)]], bufs[1 - cur], sems[1 - cur])
                    pend.wait()
                    rows = bufs[cur]

                    def row_body(r, rc):
                        for j in range(H // 16):
                            sl = pl.ds(j * 16, 16)
                            if k == 0:
                                acc[r, sl] = rows[r, sl]
                            else:
                                acc[r, sl] = acc[r, sl] + rows[r, sl]
                        return rc

                    lax.fori_loop(0, CN, row_body, 0)
                    if k + 1 < NB_NODE:
                        pend = nxt
                pltpu.sync_copy(acc, nei_hbm.at[pl.ds(base, CN)])

            return carry

        lax.fori_loop(0, (TN + NW - 1) // NW, chunk, 0)

    return sc_edge, sc_node


def kernel(fnode, fmess, agraph, bgraph, W_z, b_z, W_r, U_r, b_r, W_h, b_h, W_o, b_o):
    Wz1, Wz2 = W_z[:H], W_z[H:]
    Wh1, Wh2 = W_h[:H], W_h[H:]
    Wo1, Wo2 = W_o[:H], W_o[H:]
    bz = b_z.reshape(1, H)
    bh = b_h.reshape(1, H)
    br = b_r.reshape(1, H)
    bo = b_o.reshape(1, H)
    bg = (jnp.asarray(bgraph, jnp.int32)
          .reshape(TE, CE, NB_MESS).transpose(0, 2, 1).reshape(-1))
    ag = (jnp.asarray(agraph, jnp.int32)
          .reshape(TN, CN, NB_NODE).transpose(0, 2, 1).reshape(-1))

    sc_edge, sc_node = _make_sc_kernels()
    xz, xh, r1, hhu = _pre_call(fmess, Wz1, bz, Wh1, bh, W_r, br, U_r)
    for depth in range(1, 3):
        sh, sg = sc_edge(hhu, r1, bg)
        if depth < 2:
            hhu = _upd_hu(sh, sg, xz, xh, Wz2, Wh2, U_r)
        else:
            h = _upd_last(sh, sg, xz, xh, Wz2, Wh2, U_r)
    nei = sc_node(h, ag)
    out = _out_call(fnode, nei, Wo1, Wo2, bo)
    return out, h


# speedup vs baseline: 5.1459x; 5.1459x over previous
import functools

import jax
import jax.numpy as jnp
from jax import lax
from jax.experimental import pallas as pl
from jax.experimental.pallas import tpu as pltpu
from jax.experimental.pallas import tpu_sc as plsc

E = 160000
N_NODES = 10000
H = 128
H2 = 2 * H
NB_MESS = 6
NB_NODE = 16

NC = 2
NS = 16
NW = NC * NS

BE = 2000
BN = 1000
CE = 128
CN = 80
TE = E // CE
TN = N_NODES // CN

_f32 = jnp.float32


def _row_mask(block_rows, prog_id):
    row = lax.broadcasted_iota(jnp.int32, (block_rows, 1), 0) + prog_id * block_rows
    return (row != 0).astype(_f32)


def _pre_body(fm, wz1, bz, wh1, bh, wr, br, ur, xz_o, xh_o, r1_o, h_o, hu_o):
    fm_ = fm[...]
    xz = jnp.dot(fm_, wz1[...], preferred_element_type=_f32) + bz[...]
    xh = jnp.dot(fm_, wh1[...], preferred_element_type=_f32) + bh[...]
    r1 = jnp.dot(fm_, wr[...], preferred_element_type=_f32) + br[...]
    h1 = jax.nn.sigmoid(xz) * jnp.tanh(xh) * _row_mask(BE, pl.program_id(0))
    xz_o[...] = xz
    xh_o[...] = xh
    r1_o[...] = r1
    h_o[...] = h1
    hu_o[...] = jnp.dot(h1, ur[...], preferred_element_type=_f32)


_rows = pl.BlockSpec((BE, H), lambda i: (i, 0))
_rows2 = pl.BlockSpec((BE, H2), lambda i: (i, 0))
_full = pl.BlockSpec((H, H), lambda i: (0, 0))
_bias = pl.BlockSpec((1, H), lambda i: (0, 0))
_eSDS = jax.ShapeDtypeStruct((E, H), _f32)
_eSDS2 = jax.ShapeDtypeStruct((E, H2), _f32)

_pre_call = pl.pallas_call(
    _pre_body,
    grid=(E // BE,),
    in_specs=[_rows, _full, _bias, _full, _bias, _full, _bias, _full],
    out_specs=[_rows] * 5,
    out_shape=[_eSDS] * 5,
)


def _upd_body(want_hu, sh, sg, xz, xh, wz2, wh2, ur, *outs):
    sh_ = sh[...]
    z = jax.nn.sigmoid(xz[...] + jnp.dot(sh_, wz2[...], preferred_element_type=_f32))
    p = jnp.tanh(xh[...] + jnp.dot(sg[...], wh2[...], preferred_element_type=_f32))
    hnew = ((1.0 - z) * sh_ + z * p) * _row_mask(BE, pl.program_id(0))
    outs[0][...] = hnew
    if want_hu:
        outs[1][...] = jnp.dot(hnew, ur[...], preferred_element_type=_f32)


_upd_hu = pl.pallas_call(
    functools.partial(_upd_body, True),
    grid=(E // BE,),
    in_specs=[_rows] * 4 + [_full] * 3,
    out_specs=[_rows] * 2,
    out_shape=[_eSDS] * 2,
)

_upd_last = pl.pallas_call(
    functools.partial(_upd_body, False),
    grid=(E // BE,),
    in_specs=[_rows] * 4 + [_full] * 3,
    out_specs=_rows,
    out_shape=_eSDS,
)


def _out_body(fn, nei, wo1, wo2, bo, o):
    a = (jnp.dot(fn[...], wo1[...], preferred_element_type=_f32)
         + jnp.dot(nei[...], wo2[...], preferred_element_type=_f32) + bo[...])
    o[...] = jnp.maximum(a, 0.0) * _row_mask(BN, pl.program_id(0))


_nrows = pl.BlockSpec((BN, H), lambda i: (i, 0))
_out_call = pl.pallas_call(
    _out_body,
    grid=(N_NODES // BN,),
    in_specs=[_nrows, _nrows, _full, _full, _bias],
    out_specs=_nrows,
    out_shape=jax.ShapeDtypeStruct((N_NODES, H), _f32),
)


@functools.cache
def _make_sc_kernels():
    mesh = plsc.VectorSubcoreMesh(
        core_axis_name="c", subcore_axis_name="s", num_cores=NC, num_subcores=NS)

    @functools.partial(
        pl.kernel,
        out_type=[_eSDS, _eSDS],
        mesh=mesh,
        scratch_types=[
            pltpu.VMEM((NB_MESS * CE,), jnp.int32),
            pltpu.VMEM((CE, H), _f32),
            pltpu.VMEM((CE, H), _f32),
            pltpu.VMEM((CE, H), _f32),
            pltpu.VMEM((CE, H), _f32),
            pltpu.VMEM((CE, H), _f32),
            pltpu.VMEM((CE, H), _f32),
            pltpu.VMEM((CE, H), _f32),
            pltpu.SemaphoreType.DMA,
            pltpu.SemaphoreType.DMA,
            pltpu.SemaphoreType.DMA,
            pltpu.SemaphoreType.DMA,
            pltpu.SemaphoreType.DMA,
        ],
    )
    def sc_edge(h_hbm, hu_hbm, r1_hbm, bg_hbm, sh_hbm, sg_hbm,
                idx_v, ha, hb, ua, ub, r1v, accs, accg,
                sma, smb, sua, sub, sem_r):
        wid = lax.axis_index("s") * NC + lax.axis_index("c")
        hbufs = (ha, hb)
        ubufs = (ua, ub)
        hsems = (sma, smb)
        usems = (sua, sub)

        def chunk(i, carry):
            t = i * NW + wid

            @pl.when(t < TE)
            def _():
                base = t * CE
                pltpu.sync_copy(bg_hbm.at[pl.ds(t * (NB_MESS * CE), NB_MESS * CE)], idx_v)
                dr = pltpu.async_copy(r1_hbm.at[pl.ds(base, CE)], r1v, sem_r)
                ph = pltpu.async_copy(h_hbm.at[idx_v.at[pl.ds(0, CE)]], hbufs[0], hsems[0])
                pu = pltpu.async_copy(hu_hbm.at[idx_v.at[pl.ds(0, CE)]], ubufs[0], usems[0])
                dr.wait()
                for k in range(NB_MESS):
                    cur = k % 2
                    if k + 1 < NB_MESS:
                        isl = idx_v.at[pl.ds((k + 1) * CE, CE)]
                        nh = pltpu.async_copy(h_hbm.at[isl], hbufs[1 - cur], hsems[1 - cur])
                        nu = pltpu.async_copy(hu_hbm.at[isl], ubufs[1 - cur], usems[1 - cur])
                    ph.wait()
                    pu.wait()
                    hh = hbufs[cur]
                    uu = ubufs[cur]

                    def row_body(r, rc):
                        for j in range(H // 16):
                            sl = pl.ds(j * 16, 16)
                            hv = hh[r, sl]
                            x = r1v[r, sl] + uu[r, sl]
                            s = 1.0 / (1.0 + jnp.exp(-x))
                            if k == 0:
                                accs[r, sl] = hv
                                accg[r, sl] = s * hv
                            else:
                                accs[r, sl] = accs[r, sl] + hv
                                accg[r, sl] = accg[r, sl] + s * hv
                        return rc

                    lax.fori_loop(0, CE, row_body, 0)
                    if k + 1 < NB_MESS:
                        ph = nh
                        pu = nu
                pltpu.sync_copy(accs, sh_hbm.at[pl.ds(base, CE)])
                pltpu.sync_copy(accg, sg_hbm.at[pl.ds(base, CE)])

            return carry

        lax.fori_loop(0, (TE + NW - 1) // NW, chunk, 0)

    @functools.partial(
        pl.kernel,
        out_type=jax.ShapeDtypeStruct((N_NODES, H), _f32),
        mesh=mesh,
        scratch_types=[
            pltpu.VMEM((NB_NODE * CN,), jnp.int32),
            pltpu.VMEM((CN, H), _f32),
            pltpu.VMEM((CN, H), _f32),
            pltpu.VMEM((CN, H), _f32),
            pltpu.SemaphoreType.DMA,
            pltpu.SemaphoreType.DMA,
        ],
    )
    def sc_node(h_hbm, ag_hbm, nei_hbm, idx_v, rw0, rw1, acc, sem0, sem1):
        wid = lax.axis_index("s") * NC + lax.axis_index("c")
        bufs = (rw0, rw1)
        sems = (sem0, sem1)

        def chunk(i, carry):
            t = i * NW + wid

            @pl.when(t < TN)
            def _():
                base = t * CN
                pltpu.sync_copy(ag_hbm.at[pl.ds(t * (NB_NODE * CN), NB_NODE * CN)], idx_v)
                pend = pltpu.async_copy(h_hbm.at[idx_v.at[pl.ds(0, CN)]], bufs[0], sems[0])
                for k in range(NB_NODE):
                    cur = k % 2
                    if k + 1 < NB_NODE:
                        nxt = pltpu.async_copy(
                            h_hbm.at[idx_v.at[pl.ds((k + 1) * CN, CN)]], bufs[1 - cur], sems[1 - cur])
                    pend.wait()
                    rows = bufs[cur]

                    def row_body(r, rc):
                        for j in range(H // 16):
                            sl = pl.ds(j * 16, 16)
                            if k == 0:
                                acc[r, sl] = rows[r, sl]
                            else:
                                acc[r, sl] = acc[r, sl] + rows[r, sl]
                        return rc

                    lax.fori_loop(0, CN, row_body, 0)
                    if k + 1 < NB_NODE:
                        pend = nxt
                pltpu.sync_copy(acc, nei_hbm.at[pl.ds(base, CN)])

            return carry

        lax.fori_loop(0, (TN + NW - 1) // NW, chunk, 0)

    return sc_edge, sc_node


def kernel(fnode, fmess, agraph, bgraph, W_z, b_z, W_r, U_r, b_r, W_h, b_h, W_o, b_o):
    Wz1, Wz2 = W_z[:H], W_z[H:]
    Wh1, Wh2 = W_h[:H], W_h[H:]
    Wo1, Wo2 = W_o[:H], W_o[H:]
    bz = b_z.reshape(1, H)
    bh = b_h.reshape(1, H)
    br = b_r.reshape(1, H)
    bo = b_o.reshape(1, H)
    bg = (jnp.asarray(bgraph, jnp.int32)
          .reshape(TE, CE, NB_MESS).transpose(0, 2, 1).reshape(-1))
    ag = (jnp.asarray(agraph, jnp.int32)
          .reshape(TN, CN, NB_NODE).transpose(0, 2, 1).reshape(-1))

    sc_edge, sc_node = _make_sc_kernels()
    xz, xh, r1, h, hu = _pre_call(fmess, Wz1, bz, Wh1, bh, W_r, br, U_r)
    for depth in range(1, 3):
        sh, sg = sc_edge(h, hu, r1, bg)
        if depth < 2:
            h, hu = _upd_hu(sh, sg, xz, xh, Wz2, Wh2, U_r)
        else:
            h = _upd_last(sh, sg, xz, xh, Wz2, Wh2, U_r)
    nei = sc_node(h, ag)
    out = _out_call(fnode, nei, Wo1, Wo2, bo)
    return out, h


# speedup vs baseline: 5.3616x; 1.0419x over previous
import functools

import jax
import jax.numpy as jnp
from jax import lax
from jax.experimental import pallas as pl
from jax.experimental.pallas import tpu as pltpu
from jax.experimental.pallas import tpu_sc as plsc

E = 160000
N_NODES = 10000
H = 128
H2 = 2 * H
NB_MESS = 6
NB_NODE = 16

NC = 2
NS = 16
NW = NC * NS

BE = 2000
BN = 1000
CE = 128
CEg = 64
CN = 80
TE = E // CE
TEg = E // CEg
TN = N_NODES // CN

_f32 = jnp.float32


def _row_mask(block_rows, prog_id):
    row = lax.broadcasted_iota(jnp.int32, (block_rows, 1), 0) + prog_id * block_rows
    return (row != 0).astype(_f32)


def _pre_body(fm, wz1, bz, wh1, bh, wr, br, ur, xz_o, xh_o, r1_o, h_o, hu_o):
    fm_ = fm[...]
    xz = jnp.dot(fm_, wz1[...], preferred_element_type=_f32) + bz[...]
    xh = jnp.dot(fm_, wh1[...], preferred_element_type=_f32) + bh[...]
    r1 = jnp.dot(fm_, wr[...], preferred_element_type=_f32) + br[...]
    h1 = jax.nn.sigmoid(xz) * jnp.tanh(xh) * _row_mask(BE, pl.program_id(0))
    xz_o[...] = xz
    xh_o[...] = xh
    r1_o[...] = r1
    h_o[...] = h1
    hu_o[...] = jnp.dot(h1, ur[...], preferred_element_type=_f32)


_rows = pl.BlockSpec((BE, H), lambda i: (i, 0))
_rows2 = pl.BlockSpec((BE, H2), lambda i: (i, 0))
_full = pl.BlockSpec((H, H), lambda i: (0, 0))
_bias = pl.BlockSpec((1, H), lambda i: (0, 0))
_eSDS = jax.ShapeDtypeStruct((E, H), _f32)
_eSDS2 = jax.ShapeDtypeStruct((E, H2), _f32)

_pre_call = pl.pallas_call(
    _pre_body,
    grid=(E // BE,),
    in_specs=[_rows, _full, _bias, _full, _bias, _full, _bias, _full],
    out_specs=[_rows] * 5,
    out_shape=[_eSDS] * 5,
)


def _upd_body(want_hu, sh, sg, xz, xh, wz2, wh2, ur, *outs):
    sh_ = sh[...]
    z = jax.nn.sigmoid(xz[...] + jnp.dot(sh_, wz2[...], preferred_element_type=_f32))
    p = jnp.tanh(xh[...] + jnp.dot(sg[...], wh2[...], preferred_element_type=_f32))
    hnew = ((1.0 - z) * sh_ + z * p) * _row_mask(BE, pl.program_id(0))
    outs[0][...] = hnew
    if want_hu:
        outs[1][...] = jnp.dot(hnew, ur[...], preferred_element_type=_f32)


_upd_hu = pl.pallas_call(
    functools.partial(_upd_body, True),
    grid=(E // BE,),
    in_specs=[_rows] * 4 + [_full] * 3,
    out_specs=[_rows] * 2,
    out_shape=[_eSDS] * 2,
)

_upd_last = pl.pallas_call(
    functools.partial(_upd_body, False),
    grid=(E // BE,),
    in_specs=[_rows] * 4 + [_full] * 3,
    out_specs=_rows,
    out_shape=_eSDS,
)


def _out_body(fn, nei, wo1, wo2, bo, o):
    a = (jnp.dot(fn[...], wo1[...], preferred_element_type=_f32)
         + jnp.dot(nei[...], wo2[...], preferred_element_type=_f32) + bo[...])
    o[...] = jnp.maximum(a, 0.0) * _row_mask(BN, pl.program_id(0))


_nrows = pl.BlockSpec((BN, H), lambda i: (i, 0))
_out_call = pl.pallas_call(
    _out_body,
    grid=(N_NODES // BN,),
    in_specs=[_nrows, _nrows, _full, _full, _bias],
    out_specs=_nrows,
    out_shape=jax.ShapeDtypeStruct((N_NODES, H), _f32),
)


@functools.cache
def _make_sc_kernels():
    mesh = plsc.VectorSubcoreMesh(
        core_axis_name="c", subcore_axis_name="s", num_cores=NC, num_subcores=NS)

    @functools.partial(
        pl.kernel,
        out_type=[_eSDS, _eSDS],
        mesh=mesh,
        scratch_types=[
            pltpu.VMEM((NB_MESS * CEg,), jnp.int32),
            [pltpu.VMEM((CEg, H), _f32)] * NB_MESS,
            [pltpu.VMEM((CEg, H), _f32)] * NB_MESS,
            pltpu.VMEM((CEg, H), _f32),
            pltpu.VMEM((CEg, H), _f32),
            pltpu.VMEM((CEg, H), _f32),
            [pltpu.SemaphoreType.DMA] * NB_MESS,
            [pltpu.SemaphoreType.DMA] * NB_MESS,
            pltpu.SemaphoreType.DMA,
        ],
    )
    def sc_edge(h_hbm, hu_hbm, r1_hbm, bg_hbm, sh_hbm, sg_hbm,
                idx_v, hbufs, ubufs, r1v, accs, accg, hsems, usems, sem_r):
        wid = lax.axis_index("s") * NC + lax.axis_index("c")

        def gather_wave(k0):
            ds = []
            for k in range(k0, k0 + 3):
                isl = idx_v.at[pl.ds(k * CEg, CEg)]
                ds.append(pltpu.async_copy(h_hbm.at[isl], hbufs[k], hsems[k]))
                ds.append(pltpu.async_copy(hu_hbm.at[isl], ubufs[k], usems[k]))
            return ds

        def chunk(i, carry):
            t = i * NW + wid

            @pl.when(t < TEg)
            def _():
                base = t * CEg
                pltpu.sync_copy(
                    bg_hbm.at[pl.ds(t * (NB_MESS * CEg), NB_MESS * CEg)], idx_v)
                dr = pltpu.async_copy(r1_hbm.at[pl.ds(base, CEg)], r1v, sem_r)
                pend = gather_wave(0)
                dr.wait()
                for k0 in (0, 3):
                    if k0 + 3 < NB_MESS:
                        nxt = gather_wave(k0 + 3)
                    for d in pend:
                        d.wait()

                    def row_body(r, rc):
                        for j in range(H // 16):
                            sl = pl.ds(j * 16, 16)
                            rv = r1v[r, sl]
                            hs = None
                            gs = None
                            for k in range(k0, k0 + 3):
                                hv = hbufs[k][r, sl]
                                uv = ubufs[k][r, sl]
                                g = hv / (1.0 + jnp.exp(-(rv + uv)))
                                hs = hv if hs is None else hs + hv
                                gs = g if gs is None else gs + g
                            if k0 == 0:
                                accs[r, sl] = hs
                                accg[r, sl] = gs
                            else:
                                accs[r, sl] = accs[r, sl] + hs
                                accg[r, sl] = accg[r, sl] + gs
                        return rc

                    lax.fori_loop(0, CEg, row_body, 0)
                    if k0 + 3 < NB_MESS:
                        pend = nxt
                pltpu.sync_copy(accs, sh_hbm.at[pl.ds(base, CEg)])
                pltpu.sync_copy(accg, sg_hbm.at[pl.ds(base, CEg)])

            return carry

        lax.fori_loop(0, (TEg + NW - 1) // NW, chunk, 0)

    @functools.partial(
        pl.kernel,
        out_type=jax.ShapeDtypeStruct((N_NODES, H), _f32),
        mesh=mesh,
        scratch_types=[
            pltpu.VMEM((NB_NODE * CN,), jnp.int32),
            pltpu.VMEM((CN, H), _f32),
            pltpu.VMEM((CN, H), _f32),
            pltpu.VMEM((CN, H), _f32),
            pltpu.SemaphoreType.DMA,
            pltpu.SemaphoreType.DMA,
        ],
    )
    def sc_node(h_hbm, ag_hbm, nei_hbm, idx_v, rw0, rw1, acc, sem0, sem1):
        wid = lax.axis_index("s") * NC + lax.axis_index("c")
        bufs = (rw0, rw1)
        sems = (sem0, sem1)

        def chunk(i, carry):
            t = i * NW + wid

            @pl.when(t < TN)
            def _():
                base = t * CN
                pltpu.sync_copy(ag_hbm.at[pl.ds(t * (NB_NODE * CN), NB_NODE * CN)], idx_v)
                pend = pltpu.async_copy(h_hbm.at[idx_v.at[pl.ds(0, CN)]], bufs[0], sems[0])
                for k in range(NB_NODE):
                    cur = k % 2
                    if k + 1 < NB_NODE:
                        nxt = pltpu.async_copy(
                            h_hbm.at[idx_v.at[pl.ds((k + 1) * CN, CN)]], bufs[1 - cur], sems[1 - cur])
                    pend.wait()
                    rows = bufs[cur]

                    def row_body(r, rc):
                        for j in range(H // 16):
                            sl = pl.ds(j * 16, 16)
                            if k == 0:
                                acc[r, sl] = rows[r, sl]
                            else:
                                acc[r, sl] = acc[r, sl] + rows[r, sl]
                        return rc

                    lax.fori_loop(0, CN, row_body, 0)
                    if k + 1 < NB_NODE:
                        pend = nxt
                pltpu.sync_copy(acc, nei_hbm.at[pl.ds(base, CN)])

            return carry

        lax.fori_loop(0, (TN + NW - 1) // NW, chunk, 0)

    return sc_edge, sc_node


def kernel(fnode, fmess, agraph, bgraph, W_z, b_z, W_r, U_r, b_r, W_h, b_h, W_o, b_o):
    Wz1, Wz2 = W_z[:H], W_z[H:]
    Wh1, Wh2 = W_h[:H], W_h[H:]
    Wo1, Wo2 = W_o[:H], W_o[H:]
    bz = b_z.reshape(1, H)
    bh = b_h.reshape(1, H)
    br = b_r.reshape(1, H)
    bo = b_o.reshape(1, H)
    bg = (jnp.asarray(bgraph, jnp.int32)
          .reshape(TEg, CEg, NB_MESS).transpose(0, 2, 1).reshape(-1))
    ag = (jnp.asarray(agraph, jnp.int32)
          .reshape(TN, CN, NB_NODE).transpose(0, 2, 1).reshape(-1))

    sc_edge, sc_node = _make_sc_kernels()
    xz, xh, r1, h, hu = _pre_call(fmess, Wz1, bz, Wh1, bh, W_r, br, U_r)
    for depth in range(1, 3):
        sh, sg = sc_edge(h, hu, r1, bg)
        if depth < 2:
            h, hu = _upd_hu(sh, sg, xz, xh, Wz2, Wh2, U_r)
        else:
            h = _upd_last(sh, sg, xz, xh, Wz2, Wh2, U_r)
    nei = sc_node(h, ag)
    out = _out_call(fnode, nei, Wo1, Wo2, bo)
    return out, h
